# Initial kernel scaffold; baseline (speedup 1.0000x reference)
import jax
import jax.numpy as jnp
from jax.experimental import pallas as pl


def kernel(x, qweight, lut):
    raise NotImplementedError("write your pallas kernel here")


# speedup vs baseline: 696.3813x; 696.3813x over previous
import jax
import jax.numpy as jnp
from jax.experimental import pallas as pl
from jax.experimental.pallas import tpu as pltpu

O_BLK = 1024
W_BLK = 512


def _dequant_plane(q, shift, lut_ref):
    idx = jnp.right_shift(q, shift) & 0xF
    m0 = (idx & 1) != 0
    m1 = (idx & 2) != 0
    m2 = (idx & 4) != 0
    m3 = (idx & 8) != 0
    v = [
        jnp.where(m0, lut_ref[:, 2 * m + 1 : 2 * m + 2], lut_ref[:, 2 * m : 2 * m + 1])
        for m in range(8)
    ]
    u = [jnp.where(m1, v[2 * m + 1], v[2 * m]) for m in range(4)]
    t = [jnp.where(m2, u[2 * m + 1], u[2 * m]) for m in range(2)]
    return jnp.where(m3, t[1], t[0])


def _body(x_ref, q_ref, lut_ref, o_ref):
    wi = pl.program_id(1)

    @pl.when(wi == 0)
    def _():
        o_ref[...] = jnp.zeros_like(o_ref)

    q = q_ref[...]
    for b in range(4):
        wgt = _dequant_plane(q, 8 * b + 4, lut_ref).astype(jnp.bfloat16)
        xb = x_ref[:, b, :]
        o_ref[...] += jax.lax.dot_general(
            xb, wgt, (((1,), (1,)), ((), ())), preferred_element_type=jnp.float32
        )


def kernel(x, qweight, lut):
    B, S, K = x.shape
    O = qweight.shape[0]
    NW = K // 4
    xp = x.reshape(S, NW, 4).transpose(0, 2, 1).astype(jnp.bfloat16)
    out = pl.pallas_call(
        _body,
        grid=(O // O_BLK, NW // W_BLK),
        in_specs=[
            pl.BlockSpec((S, 4, W_BLK), lambda o, w: (0, 0, w)),
            pl.BlockSpec((O_BLK, W_BLK), lambda o, w: (o, w)),
            pl.BlockSpec((O_BLK, 16), lambda o, w: (o, 0)),
        ],
        out_specs=pl.BlockSpec((S, O_BLK), lambda o, w: (0, o)),
        out_shape=jax.ShapeDtypeStruct((S, O), jnp.float32),
        compiler_params=pltpu.CompilerParams(
            dimension_semantics=("parallel", "arbitrary"),
            vmem_limit_bytes=61 * 1024 * 1024,
        ),
        name="anyprec_linear",
    )(xp, qweight, lut)
    return out.reshape(B, S, O)


# speedup vs baseline: 1405.5181x; 2.0183x over previous
import jax
import jax.numpy as jnp
from jax.experimental import pallas as pl
from jax.experimental.pallas import tpu as pltpu

O_BLK = 1024
W_BLK = 512


def _dequant_plane(q, shift, lut_ref):
    idx = jnp.right_shift(q, shift) & 0xF
    m0 = (idx & 1) != 0
    m1 = (idx & 2) != 0
    m2 = (idx & 4) != 0
    m3 = (idx & 8) != 0
    v = [
        jnp.where(m0, lut_ref[2 * m + 1 : 2 * m + 2, :], lut_ref[2 * m : 2 * m + 1, :])
        for m in range(8)
    ]
    u = [jnp.where(m1, v[2 * m + 1], v[2 * m]) for m in range(4)]
    t = [jnp.where(m2, u[2 * m + 1], u[2 * m]) for m in range(2)]
    return jnp.where(m3, t[1], t[0])


def _body(x0_ref, x1_ref, x2_ref, x3_ref, q_ref, lut_ref, o_ref):
    wi = pl.program_id(1)

    @pl.when(wi == 0)
    def _():
        o_ref[...] = jnp.zeros_like(o_ref)

    q = q_ref[...]
    planes = [
        _dequant_plane(q, 8 * b + 4, lut_ref).astype(jnp.bfloat16) for b in range(4)
    ]
    wcat = jnp.concatenate(planes, axis=0)
    xcat = jnp.concatenate(
        [x0_ref[...], x1_ref[...], x2_ref[...], x3_ref[...]], axis=1
    )
    o_ref[...] += jax.lax.dot_general(
        xcat, wcat, (((1,), (0,)), ((), ())), preferred_element_type=jnp.float32
    )


def kernel(x, qweight, lut):
    B, S, K = x.shape
    O = qweight.shape[0]
    NW = K // 4
    NWB = NW // W_BLK
    xp = x.reshape(S, NW, 4).transpose(0, 2, 1).reshape(S, K).astype(jnp.bfloat16)
    qt = qweight.T
    lut_t = lut.T

    def x_spec(b):
        return pl.BlockSpec((S, W_BLK), lambda o, w, b=b: (0, b * NWB + w))

    out = pl.pallas_call(
        _body,
        grid=(O // O_BLK, NWB),
        in_specs=[
            x_spec(0),
            x_spec(1),
            x_spec(2),
            x_spec(3),
            pl.BlockSpec((W_BLK, O_BLK), lambda o, w: (w, o)),
            pl.BlockSpec((16, O_BLK), lambda o, w: (0, o)),
        ],
        out_specs=pl.BlockSpec((S, O_BLK), lambda o, w: (0, o)),
        out_shape=jax.ShapeDtypeStruct((S, O), jnp.float32),
        compiler_params=pltpu.CompilerParams(
            dimension_semantics=("parallel", "arbitrary"),
            vmem_limit_bytes=61 * 1024 * 1024,
        ),
        name="anyprec_linear",
    )(xp, xp, xp, xp, qt, lut_t)
    return out.reshape(B, S, O)


# speedup vs baseline: 1706.5420x; 1.2142x over previous
import jax
import jax.numpy as jnp
from jax.experimental import pallas as pl
from jax.experimental.pallas import tpu as pltpu

O_BLK = 1024
W_BLK = 512


def _body(x0_ref, x1_ref, x2_ref, x3_ref, q_ref, lut_ref, o_ref):
    wi = pl.program_id(1)

    @pl.when(wi == 0)
    def _():
        o_ref[...] = jnp.zeros_like(o_ref)

    q = q_ref[...]
    lut = lut_ref[...]
    planes = []
    for b in range(4):
        idx = jnp.right_shift(q, 8 * b + 4) & 0xF
        planes.append(jnp.take_along_axis(lut, idx, axis=1).astype(jnp.bfloat16))
    wcat = jnp.concatenate(planes, axis=1)
    xcat = jnp.concatenate(
        [x0_ref[...], x1_ref[...], x2_ref[...], x3_ref[...]], axis=1
    )
    o_ref[...] += jax.lax.dot_general(
        xcat, wcat, (((1,), (1,)), ((), ())), preferred_element_type=jnp.float32
    )


def kernel(x, qweight, lut):
    B, S, K = x.shape
    O = qweight.shape[0]
    NW = K // 4
    NWB = NW // W_BLK
    xp = x.astype(jnp.bfloat16).reshape(S, NW, 4).transpose(0, 2, 1).reshape(S, K)

    def x_spec(b):
        return pl.BlockSpec((S, W_BLK), lambda o, w, b=b: (0, b * NWB + w))

    out = pl.pallas_call(
        _body,
        grid=(O // O_BLK, NWB),
        in_specs=[
            x_spec(0),
            x_spec(1),
            x_spec(2),
            x_spec(3),
            pl.BlockSpec((O_BLK, W_BLK), lambda o, w: (o, w)),
            pl.BlockSpec((O_BLK, 16), lambda o, w: (o, 0)),
        ],
        out_specs=pl.BlockSpec((S, O_BLK), lambda o, w: (0, o)),
        out_shape=jax.ShapeDtypeStruct((S, O), jnp.float32),
        compiler_params=pltpu.CompilerParams(
            dimension_semantics=("parallel", "arbitrary"),
            vmem_limit_bytes=61 * 1024 * 1024,
        ),
        name="anyprec_linear",
    )(xp, xp, xp, xp, qweight, lut)
    return out.reshape(B, S, O)
